# baseline (speedup 1.0000x reference)
import functools

import jax
import jax.numpy as jnp
from jax import lax
from jax.experimental import pallas as pl
from jax.experimental.pallas import tpu as pltpu
from jax.experimental.pallas import tpu_sc as plsc


N_TOK = 1024
N_KEEP = 512
D = 96


def _rank_idx_body(tok_ref, w_ref, idx_ref):
    tokens = tok_ref[0, 0]
    w = w_ref[...]
    tok_r = tokens.astype(jnp.bfloat16).astype(jnp.float32)
    w_r = w.astype(jnp.bfloat16).astype(jnp.float32)
    s_col = lax.dot_general(tok_r, w_r, (((1,), (1,)), ((), ())),
                            preferred_element_type=jnp.float32)
    i_col = lax.broadcasted_iota(jnp.int32, (N_TOK, 1), 0)
    i_row = lax.broadcasted_iota(jnp.int32, (1, N_TOK), 1)
    eye = (i_col == i_row).astype(jnp.float32)
    s_row = lax.dot_general(s_col, eye, (((0,), (0,)), ((), ())),
                            preferred_element_type=jnp.float32,
                            precision=lax.Precision.HIGHEST)
    beats = (s_col > s_row) | ((s_col == s_row) & (i_col < i_row))
    rank_row = jnp.sum(beats.astype(jnp.float32), axis=0, keepdims=True)
    r_col = lax.broadcasted_iota(jnp.int32, (N_KEEP, 1), 0).astype(jnp.float32)
    sel = (rank_row == r_col).astype(jnp.float32)
    iota_col = lax.broadcasted_iota(jnp.int32, (N_TOK, 1), 0).astype(jnp.float32)
    idx = lax.dot_general(sel, iota_col, (((1,), (0,)), ((), ())),
                          preferred_element_type=jnp.float32,
                          precision=lax.Precision.HIGHEST)
    c = pl.program_id(0)
    b = pl.program_id(1)
    base = (c * 8 + b) * N_TOK
    idx_ref[0, 0] = idx.astype(jnp.int32) + base


def _first_argmax_col(m):
    mx = jnp.max(m, axis=1, keepdims=True)
    j = lax.broadcasted_iota(jnp.int32, m.shape, 1)
    return jnp.min(jnp.where(m == mx, j, m.shape[1]), axis=1, keepdims=True)


def _normalize(x):
    n = jnp.sqrt(jnp.sum(x * x, axis=1, keepdims=True))
    return x / jnp.maximum(n, 1e-8)


def _pairs_body(p1_ref, p2_ref, out_ref):
    p1 = p1_ref[0, 0][:, :D]
    p2 = p2_ref[0, 0][:, :D]
    sim = lax.dot_general(_normalize(p1), _normalize(p2),
                          (((1,), (1,)), ((), ())),
                          preferred_element_type=jnp.float32)
    top = _first_argmax_col(sim)
    j_row = lax.broadcasted_iota(jnp.int32, (N_KEEP, N_KEEP), 1)
    g1 = (top == j_row).astype(jnp.float32)
    merged = lax.dot_general(g1, (p1 + p2) * 0.5, (((1,), (0,)), ((), ())),
                             preferred_element_type=jnp.float32,
                             precision=lax.Precision.HIGHEST)
    sn = _normalize(merged)
    sim2 = lax.dot_general(sn, sn, (((1,), (1,)), ((), ())),
                           preferred_element_type=jnp.float32)
    top2 = _first_argmax_col(sim2)
    counts = jnp.sum((top2 == j_row).astype(jnp.float32), axis=0,
                     keepdims=True)
    out_ref[0, 0] = lax.dot_general(counts, merged, (((1,), (0,)), ((), ())),
                                    preferred_element_type=jnp.float32,
                                    precision=lax.Precision.HIGHEST) * (1.0 / N_KEEP)


DP = 128
_IDX_CHUNK = 128
_HALF = 512


def _make_sc_gather(n_rows, rows_per_worker):
    mesh = plsc.VectorSubcoreMesh(core_axis_name="c", subcore_axis_name="s")
    info = plsc.get_sparse_core_info()
    num_cores = info.num_cores

    @functools.partial(
        pl.kernel,
        mesh=mesh,
        out_type=jax.ShapeDtypeStruct((n_rows, DP), jnp.float32),
        scratch_types=[
            pltpu.VMEM((rows_per_worker,), jnp.int32),
            pltpu.VMEM((_HALF, DP), jnp.float32),
            pltpu.SemaphoreType.DMA,
        ],
    )
    def gather(table_hbm, idx_hbm, out_hbm, idx_v, rows_v, sem):
        wid = lax.axis_index("s") * num_cores + lax.axis_index("c")
        base = wid * rows_per_worker
        pltpu.sync_copy(idx_hbm.at[pl.ds(base, rows_per_worker)], idx_v)
        for h in range(rows_per_worker // _HALF):
            cps = [
                pltpu.async_copy(
                    table_hbm.at[idx_v.at[pl.ds(h * _HALF + j * _IDX_CHUNK,
                                                _IDX_CHUNK)]],
                    rows_v.at[pl.ds(j * _IDX_CHUNK, _IDX_CHUNK)],
                    sem,
                )
                for j in range(_HALF // _IDX_CHUNK)
            ]
            for cp in cps:
                cp.wait()
            pltpu.sync_copy(rows_v, out_hbm.at[pl.ds(base + h * _HALF, _HALF)])

    return gather


@jax.jit
def kernel(clip_embeddings_list, W_imp, b_imp):
    del b_imp
    n_clips, batch = clip_embeddings_list.shape[:2]

    idx = pl.pallas_call(
        _rank_idx_body,
        grid=(n_clips, batch),
        in_specs=[
            pl.BlockSpec((1, 1, N_TOK, D), lambda c, b: (c, b, 0, 0)),
            pl.BlockSpec((1, D), lambda c, b: (0, 0)),
        ],
        out_specs=pl.BlockSpec((1, 1, N_KEEP, 1), lambda c, b: (c, b, 0, 0)),
        out_shape=jax.ShapeDtypeStruct((n_clips, batch, N_KEEP, 1), jnp.int32),
    )(clip_embeddings_list, W_imp)

    n_rows = n_clips * batch * N_KEEP
    table = jnp.pad(clip_embeddings_list.reshape(n_clips * batch * N_TOK, D),
                    ((0, 0), (0, DP - D)))
    gather = _make_sc_gather(n_rows, n_rows // 32)
    processed = gather(table, idx.reshape(n_rows)).reshape(
        n_clips, batch, N_KEEP, DP)

    out = pl.pallas_call(
        _pairs_body,
        grid=(n_clips - 1, batch),
        in_specs=[
            pl.BlockSpec((1, 1, N_KEEP, DP), lambda p, b: (p, b, 0, 0)),
            pl.BlockSpec((1, 1, N_KEEP, DP), lambda p, b: (p + 1, b, 0, 0)),
        ],
        out_specs=pl.BlockSpec((1, 1, 1, D), lambda p, b: (p, b, 0, 0)),
        out_shape=jax.ShapeDtypeStruct((n_clips - 1, batch, 1, D), jnp.float32),
    )(processed, processed)
    return out


# speedup vs baseline: 1.0455x; 1.0455x over previous
import functools

import jax
import jax.numpy as jnp
from jax import lax
from jax.experimental import pallas as pl
from jax.experimental.pallas import tpu as pltpu
from jax.experimental.pallas import tpu_sc as plsc


N_TOK = 1024
N_KEEP = 512
D = 96


def _rank_idx_body(tok_ref, w_ref, idx_ref, pad_ref):
    tokens = tok_ref[0, 0]
    w = w_ref[...]
    tok_r = tokens.astype(jnp.bfloat16).astype(jnp.float32)
    w_r = w.astype(jnp.bfloat16).astype(jnp.float32)
    s_col = lax.dot_general(tok_r, w_r, (((1,), (1,)), ((), ())),
                            preferred_element_type=jnp.float32)
    i_col = lax.broadcasted_iota(jnp.int32, (N_TOK, 1), 0)
    i_row = lax.broadcasted_iota(jnp.int32, (1, N_TOK), 1)
    eye = (i_col == i_row).astype(jnp.float32)
    s_row = lax.dot_general(s_col, eye, (((0,), (0,)), ((), ())),
                            preferred_element_type=jnp.float32,
                            precision=lax.Precision.HIGHEST)
    beats = (s_col > s_row) | ((s_col == s_row) & (i_col < i_row))
    rank_row = jnp.sum(beats.astype(jnp.float32), axis=0, keepdims=True)
    r_col = lax.broadcasted_iota(jnp.int32, (N_KEEP, 1), 0).astype(jnp.float32)
    sel = (rank_row == r_col).astype(jnp.float32)
    iota_col = lax.broadcasted_iota(jnp.int32, (N_TOK, 1), 0).astype(jnp.float32)
    idx = lax.dot_general(sel, iota_col, (((1,), (0,)), ((), ())),
                          preferred_element_type=jnp.float32,
                          precision=lax.Precision.HIGHEST)
    c = pl.program_id(0)
    b = pl.program_id(1)
    base = (c * 8 + b) * N_TOK
    idx_ref[0, 0] = idx.astype(jnp.int32) + base
    pad_ref[0, 0] = jnp.concatenate(
        [tokens, jnp.zeros((N_TOK, DP - D), jnp.float32)], axis=1)


def _first_argmax_col(m):
    mx = jnp.max(m, axis=1, keepdims=True)
    j = lax.broadcasted_iota(jnp.int32, m.shape, 1)
    return jnp.min(jnp.where(m == mx, j, m.shape[1]), axis=1, keepdims=True)


def _normalize(x):
    n = jnp.sqrt(jnp.sum(x * x, axis=1, keepdims=True))
    return x / jnp.maximum(n, 1e-8)


def _pairs_body(p1_ref, p2_ref, out_ref):
    p1 = p1_ref[0, 0][:, :D]
    p2 = p2_ref[0, 0][:, :D]
    sim = lax.dot_general(_normalize(p1), _normalize(p2),
                          (((1,), (1,)), ((), ())),
                          preferred_element_type=jnp.float32)
    top = _first_argmax_col(sim)
    j_row = lax.broadcasted_iota(jnp.int32, (N_KEEP, N_KEEP), 1)
    g1 = (top == j_row).astype(jnp.float32)
    merged = lax.dot_general(g1, (p1 + p2) * 0.5, (((1,), (0,)), ((), ())),
                             preferred_element_type=jnp.float32,
                             precision=lax.Precision.HIGHEST)
    sn = _normalize(merged)
    sim2 = lax.dot_general(sn, sn, (((1,), (1,)), ((), ())),
                           preferred_element_type=jnp.float32)
    top2 = _first_argmax_col(sim2)
    counts = jnp.sum((top2 == j_row).astype(jnp.float32), axis=0,
                     keepdims=True)
    out_ref[0, 0] = lax.dot_general(counts, merged, (((1,), (0,)), ((), ())),
                                    preferred_element_type=jnp.float32,
                                    precision=lax.Precision.HIGHEST) * (1.0 / N_KEEP)


DP = 128
_IDX_CHUNK = 128
_HALF = 512


def _make_sc_gather(n_rows, rows_per_worker):
    mesh = plsc.VectorSubcoreMesh(core_axis_name="c", subcore_axis_name="s")
    info = plsc.get_sparse_core_info()
    num_cores = info.num_cores

    @functools.partial(
        pl.kernel,
        mesh=mesh,
        out_type=jax.ShapeDtypeStruct((n_rows, DP), jnp.float32),
        scratch_types=[
            pltpu.VMEM((rows_per_worker,), jnp.int32),
            pltpu.VMEM((_HALF, DP), jnp.float32),
            pltpu.SemaphoreType.DMA,
        ],
    )
    def gather(table_hbm, idx_hbm, out_hbm, idx_v, rows_v, sem):
        wid = lax.axis_index("s") * num_cores + lax.axis_index("c")
        base = wid * rows_per_worker
        pltpu.sync_copy(idx_hbm.at[pl.ds(base, rows_per_worker)], idx_v)
        for h in range(rows_per_worker // _HALF):
            cps = [
                pltpu.async_copy(
                    table_hbm.at[idx_v.at[pl.ds(h * _HALF + j * _IDX_CHUNK,
                                                _IDX_CHUNK)]],
                    rows_v.at[pl.ds(j * _IDX_CHUNK, _IDX_CHUNK)],
                    sem,
                )
                for j in range(_HALF // _IDX_CHUNK)
            ]
            for cp in cps:
                cp.wait()
            pltpu.sync_copy(rows_v, out_hbm.at[pl.ds(base + h * _HALF, _HALF)])

    return gather


@jax.jit
def kernel(clip_embeddings_list, W_imp, b_imp):
    del b_imp
    n_clips, batch = clip_embeddings_list.shape[:2]

    idx, table4 = pl.pallas_call(
        _rank_idx_body,
        grid=(n_clips, batch),
        in_specs=[
            pl.BlockSpec((1, 1, N_TOK, D), lambda c, b: (c, b, 0, 0)),
            pl.BlockSpec((1, D), lambda c, b: (0, 0)),
        ],
        out_specs=[
            pl.BlockSpec((1, 1, N_KEEP, 1), lambda c, b: (c, b, 0, 0)),
            pl.BlockSpec((1, 1, N_TOK, DP), lambda c, b: (c, b, 0, 0)),
        ],
        out_shape=[
            jax.ShapeDtypeStruct((n_clips, batch, N_KEEP, 1), jnp.int32),
            jax.ShapeDtypeStruct((n_clips, batch, N_TOK, DP), jnp.float32),
        ],
    )(clip_embeddings_list, W_imp)

    n_rows = n_clips * batch * N_KEEP
    table = table4.reshape(n_clips * batch * N_TOK, DP)
    gather = _make_sc_gather(n_rows, n_rows // 32)
    processed = gather(table, idx.reshape(n_rows)).reshape(
        n_clips, batch, N_KEEP, DP)

    out = pl.pallas_call(
        _pairs_body,
        grid=(n_clips - 1, batch),
        in_specs=[
            pl.BlockSpec((1, 1, N_KEEP, DP), lambda p, b: (p, b, 0, 0)),
            pl.BlockSpec((1, 1, N_KEEP, DP), lambda p, b: (p + 1, b, 0, 0)),
        ],
        out_specs=pl.BlockSpec((1, 1, 1, D), lambda p, b: (p, b, 0, 0)),
        out_shape=jax.ShapeDtypeStruct((n_clips - 1, batch, 1, D), jnp.float32),
    )(processed, processed)
    return out


# speedup vs baseline: 1.5471x; 1.4798x over previous
import functools

import jax
import jax.numpy as jnp
from jax import lax
from jax.experimental import pallas as pl
from jax.experimental.pallas import tpu as pltpu
from jax.experimental.pallas import tpu_sc as plsc


N_TOK = 1024
N_KEEP = 512
D = 96


def _rank_idx_body(tok_ref, w_ref, idx_ref, pad_ref):
    tokens = tok_ref[0, 0]
    w = w_ref[...]
    tok_r = tokens.astype(jnp.bfloat16).astype(jnp.float32)
    w_r = w.astype(jnp.bfloat16).astype(jnp.float32)
    s_col = lax.dot_general(tok_r, w_r, (((1,), (1,)), ((), ())),
                            preferred_element_type=jnp.float32)
    i_col = lax.broadcasted_iota(jnp.int32, (N_TOK, 1), 0)
    i_row = lax.broadcasted_iota(jnp.int32, (1, N_TOK), 1)
    s_row = lax.transpose(s_col, (1, 0))
    beats = (s_col > s_row) | ((s_col == s_row) & (i_col < i_row))
    rank_row = jnp.sum(beats.astype(jnp.float32), axis=0, keepdims=True)
    r_col = lax.broadcasted_iota(jnp.int32, (N_KEEP, 1), 0).astype(jnp.float32)
    sel = (rank_row == r_col).astype(jnp.float32)
    idx = jnp.sum(sel * i_row.astype(jnp.float32), axis=1,
                  keepdims=True)
    c = pl.program_id(0)
    b = pl.program_id(1)
    base = (c * 8 + b) * N_TOK
    idx_ref[0, 0] = idx.astype(jnp.int32) + base
    pad_ref[0, 0] = jnp.concatenate(
        [tokens, jnp.zeros((N_TOK, DP - D), jnp.float32)], axis=1)


def _first_argmax_col(m):
    mx = jnp.max(m, axis=1, keepdims=True)
    j = lax.broadcasted_iota(jnp.int32, m.shape, 1)
    return jnp.min(jnp.where(m == mx, j, m.shape[1]), axis=1, keepdims=True)


def _normalize(x):
    n = jnp.sqrt(jnp.sum(x * x, axis=1, keepdims=True))
    return x / jnp.maximum(n, 1e-8)


def _pairs_body(p1_ref, p2_ref, out_ref):
    p1 = p1_ref[0, 0][:, :D]
    p2 = p2_ref[0, 0][:, :D]
    sim = lax.dot_general(_normalize(p1), _normalize(p2),
                          (((1,), (1,)), ((), ())),
                          preferred_element_type=jnp.float32)
    top = _first_argmax_col(sim)
    j_row = lax.broadcasted_iota(jnp.int32, (N_KEEP, N_KEEP), 1)
    g1 = (top == j_row).astype(jnp.float32)
    x = (p1 + p2) * 0.5
    x1 = x.astype(jnp.bfloat16).astype(jnp.float32)
    r1 = x - x1
    x2 = r1.astype(jnp.bfloat16).astype(jnp.float32)
    x3 = r1 - x2
    nt = (((1,), (0,)), ((), ()))
    merged = ((lax.dot_general(g1, x1, nt, preferred_element_type=jnp.float32)
               + lax.dot_general(g1, x2, nt, preferred_element_type=jnp.float32))
              + lax.dot_general(g1, x3, nt, preferred_element_type=jnp.float32))
    sn = _normalize(merged)
    sim2 = lax.dot_general(sn, sn, (((1,), (1,)), ((), ())),
                           preferred_element_type=jnp.float32)
    top2 = _first_argmax_col(sim2)
    counts = jnp.sum((top2 == j_row).astype(jnp.float32), axis=0,
                     keepdims=True)
    out_ref[0, 0] = lax.dot_general(counts, merged, (((1,), (0,)), ((), ())),
                                    preferred_element_type=jnp.float32,
                                    precision=lax.Precision.HIGHEST) * (1.0 / N_KEEP)


DP = 128
_IDX_CHUNK = 128
_HALF = 512


def _make_sc_gather(n_rows, rows_per_worker):
    mesh = plsc.VectorSubcoreMesh(core_axis_name="c", subcore_axis_name="s")
    info = plsc.get_sparse_core_info()
    num_cores = info.num_cores

    @functools.partial(
        pl.kernel,
        mesh=mesh,
        out_type=jax.ShapeDtypeStruct((n_rows, DP), jnp.float32),
        scratch_types=[
            pltpu.VMEM((rows_per_worker,), jnp.int32),
            pltpu.VMEM((_HALF, DP), jnp.float32),
            pltpu.SemaphoreType.DMA,
        ],
    )
    def gather(table_hbm, idx_hbm, out_hbm, idx_v, rows_v, sem):
        wid = lax.axis_index("s") * num_cores + lax.axis_index("c")
        base = wid * rows_per_worker
        pltpu.sync_copy(idx_hbm.at[pl.ds(base, rows_per_worker)], idx_v)
        for h in range(rows_per_worker // _HALF):
            cps = [
                pltpu.async_copy(
                    table_hbm.at[idx_v.at[pl.ds(h * _HALF + j * _IDX_CHUNK,
                                                _IDX_CHUNK)]],
                    rows_v.at[pl.ds(j * _IDX_CHUNK, _IDX_CHUNK)],
                    sem,
                )
                for j in range(_HALF // _IDX_CHUNK)
            ]
            for cp in cps:
                cp.wait()
            pltpu.sync_copy(rows_v, out_hbm.at[pl.ds(base + h * _HALF, _HALF)])

    return gather


@jax.jit
def kernel(clip_embeddings_list, W_imp, b_imp):
    del b_imp
    n_clips, batch = clip_embeddings_list.shape[:2]

    idx, table4 = pl.pallas_call(
        _rank_idx_body,
        grid=(n_clips, batch),
        in_specs=[
            pl.BlockSpec((1, 1, N_TOK, D), lambda c, b: (c, b, 0, 0)),
            pl.BlockSpec((1, D), lambda c, b: (0, 0)),
        ],
        out_specs=[
            pl.BlockSpec((1, 1, N_KEEP, 1), lambda c, b: (c, b, 0, 0)),
            pl.BlockSpec((1, 1, N_TOK, DP), lambda c, b: (c, b, 0, 0)),
        ],
        out_shape=[
            jax.ShapeDtypeStruct((n_clips, batch, N_KEEP, 1), jnp.int32),
            jax.ShapeDtypeStruct((n_clips, batch, N_TOK, DP), jnp.float32),
        ],
    )(clip_embeddings_list, W_imp)

    n_rows = n_clips * batch * N_KEEP
    table = table4.reshape(n_clips * batch * N_TOK, DP)
    gather = _make_sc_gather(n_rows, n_rows // 32)
    processed = gather(table, idx.reshape(n_rows)).reshape(
        n_clips, batch, N_KEEP, DP)

    out = pl.pallas_call(
        _pairs_body,
        grid=(n_clips - 1, batch),
        in_specs=[
            pl.BlockSpec((1, 1, N_KEEP, DP), lambda p, b: (p, b, 0, 0)),
            pl.BlockSpec((1, 1, N_KEEP, DP), lambda p, b: (p + 1, b, 0, 0)),
        ],
        out_specs=pl.BlockSpec((1, 1, 1, D), lambda p, b: (p, b, 0, 0)),
        out_shape=jax.ShapeDtypeStruct((n_clips - 1, batch, 1, D), jnp.float32),
    )(processed, processed)
    return out


# speedup vs baseline: 1.6190x; 1.0464x over previous
import functools

import jax
import jax.numpy as jnp
from jax import lax
from jax.experimental import pallas as pl
from jax.experimental.pallas import tpu as pltpu
from jax.experimental.pallas import tpu_sc as plsc


N_TOK = 1024
N_KEEP = 512
D = 96


def _rank_idx_body(tok_ref, w_ref, idx_ref, pad_ref):
    tokens = tok_ref[0, 0]
    w = w_ref[...]
    tok_r = tokens.astype(jnp.bfloat16).astype(jnp.float32)
    w_r = w.astype(jnp.bfloat16).astype(jnp.float32)
    s_col = lax.dot_general(tok_r, w_r, (((1,), (1,)), ((), ())),
                            preferred_element_type=jnp.float32)
    i_col = lax.broadcasted_iota(jnp.int32, (N_TOK, 1), 0)
    i_row = lax.broadcasted_iota(jnp.int32, (1, N_TOK), 1)
    s_row = lax.transpose(s_col, (1, 0))
    beats = (s_col > s_row) | ((s_col == s_row) & (i_col < i_row))
    rank_row = jnp.sum(beats.astype(jnp.float32), axis=0, keepdims=True)
    r_col = lax.broadcasted_iota(jnp.int32, (N_KEEP, 1), 0).astype(jnp.float32)
    sel = (rank_row == r_col).astype(jnp.float32)
    idx = jnp.sum(sel * i_row.astype(jnp.float32), axis=1,
                  keepdims=True)
    c = pl.program_id(0)
    b = pl.program_id(1)
    base = (c * 8 + b) * N_TOK
    idx_ref[0, 0] = idx.astype(jnp.int32) + base
    pad_ref[0, 0] = jnp.concatenate(
        [tokens, jnp.zeros((N_TOK, DP - D), jnp.float32)], axis=1)


def _first_argmax_col(m):
    mx = jnp.max(m, axis=1, keepdims=True)
    j = lax.broadcasted_iota(jnp.int32, m.shape, 1)
    return jnp.min(jnp.where(m == mx, j, m.shape[1]), axis=1, keepdims=True)


def _normalize(x):
    n = jnp.sqrt(jnp.sum(x * x, axis=1, keepdims=True))
    return x / jnp.maximum(n, 1e-8)


def _pairs_body(p_ref, out_ref):
    for q in range(p_ref.shape[0] - 1):
        _pair_step(p_ref, out_ref, q)


def _pair_step(p_ref, out_ref, q):
    p1 = p_ref[q, 0][:, :D]
    p2 = p_ref[q + 1, 0][:, :D]
    sim = lax.dot_general(_normalize(p1), _normalize(p2),
                          (((1,), (1,)), ((), ())),
                          preferred_element_type=jnp.float32)
    top = _first_argmax_col(sim)
    j_row = lax.broadcasted_iota(jnp.int32, (N_KEEP, N_KEEP), 1)
    g1 = (top == j_row).astype(jnp.float32)
    x = (p1 + p2) * 0.5
    x1 = x.astype(jnp.bfloat16).astype(jnp.float32)
    r1 = x - x1
    x2 = r1.astype(jnp.bfloat16).astype(jnp.float32)
    x3 = r1 - x2
    nt = (((1,), (0,)), ((), ()))
    merged = ((lax.dot_general(g1, x1, nt, preferred_element_type=jnp.float32)
               + lax.dot_general(g1, x2, nt, preferred_element_type=jnp.float32))
              + lax.dot_general(g1, x3, nt, preferred_element_type=jnp.float32))
    sn = _normalize(merged)
    sim2 = lax.dot_general(sn, sn, (((1,), (1,)), ((), ())),
                           preferred_element_type=jnp.float32)
    top2 = _first_argmax_col(sim2)
    counts = jnp.sum((top2 == j_row).astype(jnp.float32), axis=0,
                     keepdims=True)
    out_ref[q, 0] = lax.dot_general(counts, merged, (((1,), (0,)), ((), ())),
                                    preferred_element_type=jnp.float32,
                                    precision=lax.Precision.HIGHEST) * (1.0 / N_KEEP)


DP = 128
_IDX_CHUNK = 128
_HALF = 512


def _make_sc_gather(n_rows, rows_per_worker):
    mesh = plsc.VectorSubcoreMesh(core_axis_name="c", subcore_axis_name="s")
    info = plsc.get_sparse_core_info()
    num_cores = info.num_cores

    @functools.partial(
        pl.kernel,
        mesh=mesh,
        out_type=jax.ShapeDtypeStruct((n_rows, DP), jnp.float32),
        scratch_types=[
            pltpu.VMEM((rows_per_worker,), jnp.int32),
            pltpu.VMEM((_HALF, DP), jnp.float32),
            pltpu.SemaphoreType.DMA,
        ],
    )
    def gather(table_hbm, idx_hbm, out_hbm, idx_v, rows_v, sem):
        wid = lax.axis_index("s") * num_cores + lax.axis_index("c")
        base = wid * rows_per_worker
        pltpu.sync_copy(idx_hbm.at[pl.ds(base, rows_per_worker)], idx_v)
        for h in range(rows_per_worker // _HALF):
            cps = [
                pltpu.async_copy(
                    table_hbm.at[idx_v.at[pl.ds(h * _HALF + j * _IDX_CHUNK,
                                                _IDX_CHUNK)]],
                    rows_v.at[pl.ds(j * _IDX_CHUNK, _IDX_CHUNK)],
                    sem,
                )
                for j in range(_HALF // _IDX_CHUNK)
            ]
            for cp in cps:
                cp.wait()
            pltpu.sync_copy(rows_v, out_hbm.at[pl.ds(base + h * _HALF, _HALF)])

    return gather


@jax.jit
def kernel(clip_embeddings_list, W_imp, b_imp):
    del b_imp
    n_clips, batch = clip_embeddings_list.shape[:2]

    idx, table4 = pl.pallas_call(
        _rank_idx_body,
        grid=(n_clips, batch),
        in_specs=[
            pl.BlockSpec((1, 1, N_TOK, D), lambda c, b: (c, b, 0, 0)),
            pl.BlockSpec((1, D), lambda c, b: (0, 0)),
        ],
        out_specs=[
            pl.BlockSpec((1, 1, N_KEEP, 1), lambda c, b: (c, b, 0, 0)),
            pl.BlockSpec((1, 1, N_TOK, DP), lambda c, b: (c, b, 0, 0)),
        ],
        out_shape=[
            jax.ShapeDtypeStruct((n_clips, batch, N_KEEP, 1), jnp.int32),
            jax.ShapeDtypeStruct((n_clips, batch, N_TOK, DP), jnp.float32),
        ],
    )(clip_embeddings_list, W_imp)

    n_rows = n_clips * batch * N_KEEP
    table = table4.reshape(n_clips * batch * N_TOK, DP)
    gather = _make_sc_gather(n_rows, n_rows // 32)
    processed = gather(table, idx.reshape(n_rows)).reshape(
        n_clips, batch, N_KEEP, DP)

    out = pl.pallas_call(
        _pairs_body,
        grid=(batch,),
        in_specs=[
            pl.BlockSpec((n_clips, 1, N_KEEP, DP), lambda b: (0, b, 0, 0)),
        ],
        out_specs=pl.BlockSpec((n_clips - 1, 1, 1, D), lambda b: (0, b, 0, 0)),
        out_shape=jax.ShapeDtypeStruct((n_clips - 1, batch, 1, D), jnp.float32),
    )(processed)
    return out


# speedup vs baseline: 1.6790x; 1.0371x over previous
import functools

import jax
import jax.numpy as jnp
from jax import lax
from jax.experimental import pallas as pl
from jax.experimental.pallas import tpu as pltpu
from jax.experimental.pallas import tpu_sc as plsc


N_TOK = 1024
N_KEEP = 512
D = 96


def _rank_idx_body(tok_ref, w_ref, idx_ref, pad_ref):
    for b in range(tok_ref.shape[1]):
        _rank_idx_step(tok_ref, w_ref, idx_ref, pad_ref, b)


def _rank_idx_step(tok_ref, w_ref, idx_ref, pad_ref, b):
    tokens = tok_ref[0, b]
    w = w_ref[...]
    tok_r = tokens.astype(jnp.bfloat16).astype(jnp.float32)
    w_r = w.astype(jnp.bfloat16).astype(jnp.float32)
    s_col = lax.dot_general(tok_r, w_r, (((1,), (1,)), ((), ())),
                            preferred_element_type=jnp.float32)
    i_col = lax.broadcasted_iota(jnp.int32, (N_TOK, 1), 0)
    i_row = lax.broadcasted_iota(jnp.int32, (1, N_TOK), 1)
    s_row = lax.transpose(s_col, (1, 0))
    beats = (s_col > s_row) | ((s_col == s_row) & (i_col < i_row))
    rank_row = jnp.sum(beats.astype(jnp.float32), axis=0, keepdims=True)
    r_col = lax.broadcasted_iota(jnp.int32, (N_KEEP, 1), 0).astype(jnp.float32)
    sel = (rank_row == r_col).astype(jnp.float32)
    idx = jnp.sum(sel * i_row.astype(jnp.float32), axis=1,
                  keepdims=True)
    c = pl.program_id(0)
    base = (c * 8 + b) * N_TOK
    idx_ref[0, b] = idx.astype(jnp.int32) + base
    pad_ref[0, b, :, :D] = tokens


def _first_argmax_col(m):
    mx = jnp.max(m, axis=1, keepdims=True)
    j = lax.broadcasted_iota(jnp.int32, m.shape, 1)
    return jnp.min(jnp.where(m == mx, j, m.shape[1]), axis=1, keepdims=True)


def _normalize(x):
    n = jnp.sqrt(jnp.sum(x * x, axis=1, keepdims=True))
    return x / jnp.maximum(n, 1e-8)


def _pairs_body(p_ref, out_ref):
    for q in range(p_ref.shape[0] - 1):
        _pair_step(p_ref, out_ref, q)


def _pair_step(p_ref, out_ref, q):
    p1 = p_ref[q, 0][:, :D]
    p2 = p_ref[q + 1, 0][:, :D]
    sim = lax.dot_general(_normalize(p1), _normalize(p2),
                          (((1,), (1,)), ((), ())),
                          preferred_element_type=jnp.float32)
    top = _first_argmax_col(sim)
    j_row = lax.broadcasted_iota(jnp.int32, (N_KEEP, N_KEEP), 1)
    g1 = (top == j_row).astype(jnp.float32)
    x = (p1 + p2) * 0.5
    x1 = x.astype(jnp.bfloat16).astype(jnp.float32)
    r1 = x - x1
    x2 = r1.astype(jnp.bfloat16).astype(jnp.float32)
    x3 = r1 - x2
    nt = (((1,), (0,)), ((), ()))
    merged = ((lax.dot_general(g1, x1, nt, preferred_element_type=jnp.float32)
               + lax.dot_general(g1, x2, nt, preferred_element_type=jnp.float32))
              + lax.dot_general(g1, x3, nt, preferred_element_type=jnp.float32))
    sn = _normalize(merged)
    sim2 = lax.dot_general(sn, sn, (((1,), (1,)), ((), ())),
                           preferred_element_type=jnp.float32)
    top2 = _first_argmax_col(sim2)
    counts = jnp.sum((top2 == j_row).astype(jnp.float32), axis=0,
                     keepdims=True)
    out_ref[q, 0] = lax.dot_general(counts, merged, (((1,), (0,)), ((), ())),
                                    preferred_element_type=jnp.float32,
                                    precision=lax.Precision.HIGHEST) * (1.0 / N_KEEP)


DP = 128
_IDX_CHUNK = 128
_HALF = 512


def _make_sc_gather(n_rows, rows_per_worker):
    mesh = plsc.VectorSubcoreMesh(core_axis_name="c", subcore_axis_name="s")
    info = plsc.get_sparse_core_info()
    num_cores = info.num_cores

    @functools.partial(
        pl.kernel,
        mesh=mesh,
        out_type=jax.ShapeDtypeStruct((n_rows, DP), jnp.float32),
        scratch_types=[
            pltpu.VMEM((rows_per_worker,), jnp.int32),
            pltpu.VMEM((_HALF, DP), jnp.float32),
            pltpu.SemaphoreType.DMA,
        ],
    )
    def gather(table_hbm, idx_hbm, out_hbm, idx_v, rows_v, sem):
        wid = lax.axis_index("s") * num_cores + lax.axis_index("c")
        base = wid * rows_per_worker
        pltpu.sync_copy(idx_hbm.at[pl.ds(base, rows_per_worker)], idx_v)
        for h in range(rows_per_worker // _HALF):
            cps = [
                pltpu.async_copy(
                    table_hbm.at[idx_v.at[pl.ds(h * _HALF + j * _IDX_CHUNK,
                                                _IDX_CHUNK)]],
                    rows_v.at[pl.ds(j * _IDX_CHUNK, _IDX_CHUNK)],
                    sem,
                )
                for j in range(_HALF // _IDX_CHUNK)
            ]
            for cp in cps:
                cp.wait()
            pltpu.sync_copy(rows_v, out_hbm.at[pl.ds(base + h * _HALF, _HALF)])

    return gather


@jax.jit
def kernel(clip_embeddings_list, W_imp, b_imp):
    del b_imp
    n_clips, batch = clip_embeddings_list.shape[:2]

    idx, table4 = pl.pallas_call(
        _rank_idx_body,
        grid=(n_clips,),
        in_specs=[
            pl.BlockSpec((1, batch, N_TOK, D), lambda c: (c, 0, 0, 0)),
            pl.BlockSpec((1, D), lambda c: (0, 0)),
        ],
        out_specs=[
            pl.BlockSpec((1, batch, N_KEEP, 1), lambda c: (c, 0, 0, 0)),
            pl.BlockSpec((1, batch, N_TOK, DP), lambda c: (c, 0, 0, 0)),
        ],
        out_shape=[
            jax.ShapeDtypeStruct((n_clips, batch, N_KEEP, 1), jnp.int32),
            jax.ShapeDtypeStruct((n_clips, batch, N_TOK, DP), jnp.float32),
        ],
    )(clip_embeddings_list, W_imp)

    n_rows = n_clips * batch * N_KEEP
    table = table4.reshape(n_clips * batch * N_TOK, DP)
    gather = _make_sc_gather(n_rows, n_rows // 32)
    processed = gather(table, idx.reshape(n_rows)).reshape(
        n_clips, batch, N_KEEP, DP)

    out = pl.pallas_call(
        _pairs_body,
        grid=(batch,),
        in_specs=[
            pl.BlockSpec((n_clips, 1, N_KEEP, DP), lambda b: (0, b, 0, 0)),
        ],
        out_specs=pl.BlockSpec((n_clips - 1, 1, 1, D), lambda b: (0, b, 0, 0)),
        out_shape=jax.ShapeDtypeStruct((n_clips - 1, batch, 1, D), jnp.float32),
    )(processed)
    return out
